# baseline (speedup 1.0000x reference)
import functools

import jax
import jax.numpy as jnp
from jax import lax
from jax.experimental import pallas as pl
from jax.experimental.pallas import tpu as pltpu
from jax.experimental.pallas import tpu_sc as plsc

MAX_MEL = 2000
B, S, H = 16, 512, 256
HALF = MAX_MEL // 2
NCHUNK = 16
CHUNK = 64
TAIL = HALF - (NCHUNK - 1) * CHUNK
PADW = NCHUNK * CHUNK
NLANE = 16
RING = 6


def _lr_body(x_hbm, dur_hbm, out_hbm, mel_hbm,
             dur_v, cums_v, hist_v, idxbuf_v,
             rows_bufs, mel_v, gsems, wsems):
    c = lax.axis_index("c")
    s = lax.axis_index("s")
    w = c * 16 + s
    b = w >> 1
    half = w & 1
    mlo = half * HALF

    pltpu.sync_copy(dur_hbm.at[pl.ds(b * S, S)], dur_v)
    carry = jnp.int32(0)
    base = jnp.int32(0)
    for i in range(S // NLANE):
        v = dur_v[pl.ds(i * NLANE, NLANE)]
        cv = plsc.cumsum(v) + carry
        cums_v[pl.ds(i * NLANE, NLANE)] = cv
        carry = carry + jnp.sum(v)
        base = base + jnp.sum((cv < mlo).astype(jnp.int32))
    cums_v[pl.ds(S, NLANE)] = jnp.full((NLANE,), -1, jnp.int32)

    zeros = jnp.zeros((NLANE,), jnp.int32)
    for j in range(PADW // NLANE):
        hist_v[pl.ds(j * NLANE, NLANE)] = zeros
    lane = lax.iota(jnp.int32, NLANE)
    for i in range(S // NLANE):
        cur = cums_v[pl.ds(i * NLANE, NLANE)]
        nxt = cums_v[pl.ds(i * NLANE + 1, NLANE)]
        pos = cur - mlo
        msk = (nxt != cur) & (pos >= 0) & (pos < PADW)
        plsc.store_scatter(hist_v, [pos], lane + (i * NLANE + 1), mask=msk)

    run = base
    rowbase = b * S
    for j in range(PADW // NLANE):
        v = hist_v[pl.ds(j * NLANE, NLANE)]
        cm = jnp.maximum(plsc.cummax(v), run)
        run = jnp.max(cm)
        idxbuf_v[j // (CHUNK // NLANE),
                 pl.ds((j % (CHUNK // NLANE)) * NLANE, NLANE)] = (
            jnp.minimum(cm, S - 1) + rowbase)

    @pl.when((w & 15) == 0)
    def _mel():
        gb = (w >> 4) * 8
        mel_vec = jnp.zeros((NLANE,), jnp.int32)
        for bb in range(8):
            pltpu.sync_copy(dur_hbm.at[pl.ds((gb + bb) * S, S)], dur_v)
            acc = jnp.zeros((NLANE,), jnp.int32)
            for i in range(S // NLANE):
                acc = acc + dur_v[pl.ds(i * NLANE, NLANE)]
            t = jnp.minimum(jnp.sum(acc), MAX_MEL)
            mel_vec = jnp.where(lane == bb, t, mel_vec)
        mel_v[...] = mel_vec
        pltpu.sync_copy(mel_v.at[pl.ds(0, 8)], mel_hbm.at[pl.ds(gb, 8)])

    gout = b * MAX_MEL + mlo
    gh = [None] * NCHUNK
    wh = [None] * NCHUNK

    def start_gather(j):
        nrows = CHUNK if j + 1 < NCHUNK else TAIL
        gh[j] = pltpu.async_copy(
            x_hbm.at[idxbuf_v.at[j, pl.ds(0, nrows)]],
            rows_bufs[j % RING].at[pl.ds(0, nrows)], gsems[j % RING])

    prime = RING - 2
    for j in range(prime):
        start_gather(j)
    for j in range(NCHUNK):
        if j + prime < NCHUNK:
            if j + prime - RING >= 0:
                wh[j + prime - RING].wait()
            start_gather(j + prime)
        gh[j].wait()
        nrows = CHUNK if j + 1 < NCHUNK else TAIL
        wh[j] = pltpu.async_copy(rows_bufs[j % RING].at[pl.ds(0, nrows)],
                                 out_hbm.at[pl.ds(gout + j * CHUNK, nrows)],
                                 wsems[j % RING])
    for j in range(max(0, NCHUNK - RING), NCHUNK):
        wh[j].wait()


@functools.partial(
    pl.kernel,
    out_type=(jax.ShapeDtypeStruct((B * MAX_MEL, H), jnp.float32),
              jax.ShapeDtypeStruct((B,), jnp.int32)),
    mesh=plsc.VectorSubcoreMesh(core_axis_name="c", subcore_axis_name="s"),
    scratch_types=(
        pltpu.VMEM((S,), jnp.int32),
        pltpu.VMEM((S + NLANE,), jnp.int32),
        pltpu.VMEM((PADW,), jnp.int32),
        pltpu.VMEM((NCHUNK, CHUNK), jnp.int32),
        *[pltpu.VMEM((CHUNK, H), jnp.float32) for _ in range(RING)],
        pltpu.VMEM((NLANE,), jnp.int32),
        *[pltpu.SemaphoreType.DMA for _ in range(2 * RING)],
    ),
    compiler_params=pltpu.CompilerParams(needs_layout_passes=False),
)
def _lr_kernel(x_hbm, dur_hbm, out_hbm, mel_hbm, *scratch):
    dur_v, cums_v, hist_v, idxbuf_v = scratch[0:4]
    rows_bufs = scratch[4:4 + RING]
    mel_v = scratch[4 + RING]
    gsems = scratch[5 + RING:5 + 2 * RING]
    wsems = scratch[5 + 2 * RING:5 + 3 * RING]
    _lr_body(x_hbm, dur_hbm, out_hbm, mel_hbm,
             dur_v, cums_v, hist_v, idxbuf_v, rows_bufs, mel_v, gsems, wsems)


def kernel(x, duration, max_len):
    del max_len
    out_flat, mel_len = _lr_kernel(x.reshape(B * S, H), duration.reshape(B * S))
    return out_flat.reshape(B, MAX_MEL, H), mel_len
